# baseline (speedup 1.0000x reference)
import functools

import jax
import jax.numpy as jnp
from jax import lax
from jax.experimental import pallas as pl
from jax.experimental.pallas import tpu as pltpu
from jax.experimental.pallas import tpu_sc as plsc

VOCAB = 100000
EMBED = 64
BATCH = 1024
CTX = 20

NC = 2
NS = 16
L = 16
NW = NC * NS
B_PER_W = BATCH // NW
IDX_PER_W = B_PER_W * CTX
CHUNK = 128
NCHUNK = IDX_PER_W // CHUNK
PAIR = 128

RBLK = 1024
_NRB = (VOCAB + RBLK - 1) // RBLK
_NPB = (_NRB + 1) // 2
KSPLIT = _NPB * RBLK
NPAIR = KSPLIT

_SC_MESH = plsc.VectorSubcoreMesh(core_axis_name="c", subcore_axis_name="s")


@functools.partial(
    pl.kernel,
    out_type=jax.ShapeDtypeStruct((BATCH, EMBED), jnp.float32),
    mesh=_SC_MESH,
    scratch_types=[
        pltpu.VMEM((IDX_PER_W,), jnp.int32),
        pltpu.VMEM((IDX_PER_W * L,), jnp.int32),
        pltpu.VMEM((IDX_PER_W, PAIR), jnp.float32),
        pltpu.VMEM((B_PER_W, EMBED), jnp.float32),
        pltpu.SemaphoreType.DMA,
    ],
    compiler_params=pltpu.CompilerParams(needs_layout_passes=False),
)
def _gather_pool(embp_hbm, gidx_hbm, pcol_hbm, pooled_hbm,
                 idx_v, pcol_v, rows_v, pooled_v, sem):
    wid = lax.axis_index("s") * NC + lax.axis_index("c")
    base_b = wid * B_PER_W

    pltpu.sync_copy(gidx_hbm.at[wid], idx_v)
    pltpu.sync_copy(pcol_hbm.at[wid], pcol_v)
    copies = [
        pltpu.async_copy(
            embp_hbm.at[idx_v.at[pl.ds(j * CHUNK, CHUNK)]],
            rows_v.at[pl.ds(j * CHUNK, CHUNK)],
            sem,
        )
        for j in range(NCHUNK)
    ]
    for c in copies:
        c.wait()

    def b_body(b, carry):
        accs = [jnp.zeros((L,), jnp.float32) for _ in range(EMBED // L)]
        for c in range(CTX):
            r = b * CTX + c
            row_vec = jnp.full((L,), r, jnp.int32)
            col0 = pcol_v[pl.ds(r * L, L)]
            for d in range(EMBED // L):
                accs[d] = accs[d] + plsc.load_gather(
                    rows_v, [row_vec, col0 + (d * L)]
                )
        for d in range(EMBED // L):
            pooled_v[b, pl.ds(d * L, L)] = accs[d] * (1.0 / CTX)
        return carry

    lax.fori_loop(0, B_PER_W, b_body, 0)
    pltpu.sync_copy(pooled_v, pooled_hbm.at[pl.ds(base_b, B_PER_W)])


def _repack_body(wl_ref, wr_ref, o_ref):
    o_ref[:, 0:EMBED] = jnp.swapaxes(wl_ref[...], 0, 1)
    o_ref[:, EMBED:PAIR] = jnp.swapaxes(wr_ref[...], 0, 1)


_repack = pl.pallas_call(
    _repack_body,
    grid=(_NPB,),
    in_specs=[
        pl.BlockSpec((EMBED, RBLK), lambda i: (0, i)),
        pl.BlockSpec((EMBED, RBLK), lambda i: (0, i + _NPB)),
    ],
    out_specs=pl.BlockSpec((RBLK, PAIR), lambda i: (i, 0)),
    out_shape=jax.ShapeDtypeStruct((NPAIR, PAIR), jnp.float32),
)


VBLK = 4096
_NVB = (VOCAB + VBLK - 1) // VBLK


def _proj_body(w_ref, p_ref, o_ref):
    o_ref[...] = lax.dot_general(
        w_ref[...],
        p_ref[...],
        (((0,), (1,)), ((), ())),
        preferred_element_type=jnp.float32,
    )


_proj = pl.pallas_call(
    _proj_body,
    grid=(_NVB,),
    in_specs=[
        pl.BlockSpec((EMBED, VBLK), lambda i: (0, i)),
        pl.BlockSpec((BATCH, EMBED), lambda i: (0, 0)),
    ],
    out_specs=pl.BlockSpec((VBLK, BATCH), lambda i: (i, 0)),
    out_shape=jax.ShapeDtypeStruct((VOCAB, BATCH), jnp.float32),
)


def kernel(x, emb_table, ffw_w):
    wview = emb_table.T
    embp = _repack(wview, wview)
    xf = x.reshape(NW, IDX_PER_W)
    hi = (xf >= KSPLIT).astype(jnp.int32)
    gidx = xf - hi * KSPLIT
    pcol = (hi * EMBED)[:, :, None] + jnp.arange(L, dtype=jnp.int32)
    pcol = pcol.reshape(NW, IDX_PER_W * L)
    pooled = _gather_pool(embp, gidx, pcol)
    return _proj(ffw_w.T, pooled).T


# speedup vs baseline: 1.1348x; 1.1348x over previous
import functools

import jax
import jax.numpy as jnp
from jax import lax
from jax.experimental import pallas as pl
from jax.experimental.pallas import tpu as pltpu
from jax.experimental.pallas import tpu_sc as plsc

VOCAB = 100000
EMBED = 64
BATCH = 1024
CTX = 20

NC = 2
NS = 16
L = 16
NW = NC * NS
B_PER_W = BATCH // NW
IDX_PER_W = B_PER_W * CTX
CHUNK = 128
NCHUNK = IDX_PER_W // CHUNK
PAD = 128

_SC_MESH = plsc.VectorSubcoreMesh(core_axis_name="c", subcore_axis_name="s")


@functools.partial(
    pl.kernel,
    out_type=jax.ShapeDtypeStruct((BATCH, EMBED), jnp.float32),
    mesh=_SC_MESH,
    scratch_types=[
        pltpu.VMEM((IDX_PER_W,), jnp.int32),
        pltpu.VMEM((IDX_PER_W, PAD), jnp.float32),
        pltpu.VMEM((B_PER_W, EMBED), jnp.float32),
        pltpu.SemaphoreType.DMA,
    ],
    compiler_params=pltpu.CompilerParams(needs_layout_passes=False),
)
def _gather_pool(embp_hbm, xf_hbm, pooled_hbm, idx_v, rows_v, pooled_v, sem):
    wid = lax.axis_index("s") * NC + lax.axis_index("c")
    base_b = wid * B_PER_W

    pltpu.sync_copy(xf_hbm.at[wid], idx_v)
    copies = [
        pltpu.async_copy(
            embp_hbm.at[idx_v.at[pl.ds(j * CHUNK, CHUNK)]],
            rows_v.at[pl.ds(j * CHUNK, CHUNK)],
            sem,
        )
        for j in range(NCHUNK)
    ]
    for c in copies:
        c.wait()

    def b_body(b, carry):
        for d in range(EMBED // L):
            acc = rows_v[b * CTX, pl.ds(d * L, L)]
            for c in range(1, CTX):
                acc = acc + rows_v[b * CTX + c, pl.ds(d * L, L)]
            pooled_v[b, pl.ds(d * L, L)] = acc * (1.0 / CTX)
        return carry

    lax.fori_loop(0, B_PER_W, b_body, 0)
    pltpu.sync_copy(pooled_v, pooled_hbm.at[pl.ds(base_b, B_PER_W)])


RBLK = 16384
_NRB = (VOCAB + RBLK - 1) // RBLK


def _repack_body(w_ref, o_ref):
    o_ref[:, 0:EMBED] = jnp.swapaxes(w_ref[...], 0, 1)


_repack = pl.pallas_call(
    _repack_body,
    grid=(_NRB,),
    in_specs=[pl.BlockSpec((EMBED, RBLK), lambda i: (0, i))],
    out_specs=pl.BlockSpec((RBLK, PAD), lambda i: (i, 0)),
    out_shape=jax.ShapeDtypeStruct((VOCAB, PAD), jnp.float32),
)


VBLK = 4096
_NVB = (VOCAB + VBLK - 1) // VBLK


def _proj_body(w_ref, p_ref, o_ref):
    o_ref[...] = lax.dot_general(
        w_ref[...],
        p_ref[...],
        (((0,), (1,)), ((), ())),
        preferred_element_type=jnp.float32,
    )


_proj = pl.pallas_call(
    _proj_body,
    grid=(_NVB,),
    in_specs=[
        pl.BlockSpec((EMBED, VBLK), lambda i: (0, i)),
        pl.BlockSpec((BATCH, EMBED), lambda i: (0, 0)),
    ],
    out_specs=pl.BlockSpec((VBLK, BATCH), lambda i: (i, 0)),
    out_shape=jax.ShapeDtypeStruct((VOCAB, BATCH), jnp.float32),
)


def kernel(x, emb_table, ffw_w):
    embp = _repack(emb_table.T)
    xf = x.reshape(NW, IDX_PER_W)
    pooled = _gather_pool(embp, xf)
    return _proj(ffw_w.T, pooled).T


# speedup vs baseline: 1.1350x; 1.0002x over previous
import functools

import jax
import jax.numpy as jnp
from jax import lax
from jax.experimental import pallas as pl
from jax.experimental.pallas import tpu as pltpu
from jax.experimental.pallas import tpu_sc as plsc

VOCAB = 100000
EMBED = 64
BATCH = 1024
CTX = 20

NC = 2
NS = 16
L = 16
NW = NC * NS
B_PER_W = BATCH // NW
IDX_PER_W = B_PER_W * CTX
CHUNK = 128
NCHUNK = IDX_PER_W // CHUNK
PAD = 128

_SC_MESH = plsc.VectorSubcoreMesh(core_axis_name="c", subcore_axis_name="s")


@functools.partial(
    pl.kernel,
    out_type=jax.ShapeDtypeStruct((BATCH, EMBED), jnp.float32),
    mesh=_SC_MESH,
    scratch_types=[
        pltpu.VMEM((IDX_PER_W,), jnp.int32),
        pltpu.VMEM((IDX_PER_W, PAD), jnp.float32),
        pltpu.VMEM((B_PER_W, EMBED), jnp.float32),
        pltpu.SemaphoreType.DMA,
    ],
    compiler_params=pltpu.CompilerParams(needs_layout_passes=False),
)
def _gather_pool(embp_hbm, xf_hbm, pooled_hbm, idx_v, rows_v, pooled_v, sem):
    wid = lax.axis_index("s") * NC + lax.axis_index("c")
    base_b = wid * B_PER_W

    pltpu.sync_copy(xf_hbm.at[wid], idx_v)
    copies = [
        pltpu.async_copy(
            embp_hbm.at[idx_v.at[pl.ds(j * CHUNK, CHUNK)]],
            rows_v.at[pl.ds(j * CHUNK, CHUNK)],
            sem,
        )
        for j in range(NCHUNK)
    ]
    for c in copies:
        c.wait()

    def b_body(b, carry):
        for d in range(EMBED // L):
            acc = rows_v[b * CTX, pl.ds(d * L, L)]
            for c in range(1, CTX):
                acc = acc + rows_v[b * CTX + c, pl.ds(d * L, L)]
            pooled_v[b, pl.ds(d * L, L)] = acc * (1.0 / CTX)
        return carry

    lax.fori_loop(0, B_PER_W, b_body, 0)
    pltpu.sync_copy(pooled_v, pooled_hbm.at[pl.ds(base_b, B_PER_W)])


RBLK = 16384
_NRB = (VOCAB + RBLK - 1) // RBLK


def _repack_body(w_ref, o_ref):
    o_ref[:, 0:EMBED] = jnp.swapaxes(w_ref[...], 0, 1)


_repack = pl.pallas_call(
    _repack_body,
    grid=(_NRB,),
    in_specs=[pl.BlockSpec((EMBED, RBLK), lambda i: (0, i))],
    out_specs=pl.BlockSpec((RBLK, PAD), lambda i: (i, 0)),
    out_shape=jax.ShapeDtypeStruct((VOCAB, PAD), jnp.float32),
)


VBLK = 6144
_NVB = (VOCAB + VBLK - 1) // VBLK


def _proj_body(w_ref, p_ref, o_ref):
    o_ref[...] = lax.dot_general(
        w_ref[...],
        p_ref[...],
        (((0,), (1,)), ((), ())),
        preferred_element_type=jnp.float32,
    )


_proj = pl.pallas_call(
    _proj_body,
    grid=(_NVB,),
    in_specs=[
        pl.BlockSpec((EMBED, VBLK), lambda i: (0, i)),
        pl.BlockSpec((BATCH, EMBED), lambda i: (0, 0)),
    ],
    out_specs=pl.BlockSpec((VBLK, BATCH), lambda i: (i, 0)),
    out_shape=jax.ShapeDtypeStruct((VOCAB, BATCH), jnp.float32),
)


def kernel(x, emb_table, ffw_w):
    embp = _repack(emb_table.T)
    xf = x.reshape(NW, IDX_PER_W)
    pooled = _gather_pool(embp, xf)
    return _proj(ffw_w.T, pooled).T
